# baseline (speedup 1.0000x reference)
import dataclasses
import functools

import jax
import jax.numpy as jnp
from jax import lax
from jax.experimental import pallas as pl
from jax.experimental.pallas import tpu as pltpu
from jax.experimental.pallas import tpu_sc as plsc

T = 4
N = 10000
E = 320000
D = 128
H = 128
NH = 4
HD = H // NH
M = T * N

NS = 16
NCORE = 2
KH = H // 2
EC = E // NS
B = 80
ITER = 252
ECP = ITER * B
NBUF = 6
WS = 2
CW = 16
CR = 10240 // CW
NP = 10240
RPT = NP // NS
CPT = CR // NS

_mesh = plsc.VectorSubcoreMesh(core_axis_name="c", subcore_axis_name="s")


def _sc_agg_body(with_cnt, *refs):
    if with_cnt:
        (y_hbm, src_hbm, dst_hbm, z_hbm, zc_hbm, idc_hbm, out_hbm, cnt_hbm,
         src_v, dst_v, *bufs, acc, gsem, ssem, cl_v, idc_v, scnt) = refs
    else:
        (y_hbm, src_hbm, dst_hbm, z_hbm, out_hbm,
         src_v, dst_v, *bufs, acc, gsem, ssem) = refs

    c = lax.axis_index("c")
    s = lax.axis_index("s")

    if with_cnt:
        pltpu.sync_copy(idc_hbm, idc_v)

    for t in range(T):
        mine = (t // 2) == c
        pltpu.sync_copy(src_hbm.at[c, t, s], src_v)
        pltpu.sync_copy(dst_hbm.at[t, s], dst_v)
        pltpu.sync_copy(z_hbm, acc.at[pl.ds(s * RPT, RPT)])
        if with_cnt:
            @pl.when(mine)
            def _():
                pltpu.sync_copy(zc_hbm, cl_v)
                pltpu.sync_copy(zc_hbm.at[pl.ds(0, CPT)],
                                scnt.at[pl.ds(s * CPT, CPT)])
        plsc.subcore_barrier()

        for k in range(NBUF - WS):
            pltpu.make_async_copy(y_hbm.at[src_v.at[k]], bufs[k],
                                  gsem).start()

        @pl.loop(0, ITER, step=NBUF)
        def _(i):
            for k in range(NBUF):
                j = i + k
                pltpu.make_async_copy(y_hbm.at[src_v.at[j]], bufs[k],
                                      gsem).wait()
                pltpu.async_copy(bufs[k], acc.at[dst_v.at[j]], ssem,
                                 add=True)

                @pl.when(j >= WS)
                def _():
                    pltpu.make_async_copy(bufs[k], acc.at[dst_v.at[j]],
                                          ssem).wait()

                @pl.when(j + NBUF - WS < ITER)
                def _():
                    kk = (k + NBUF - WS) % NBUF
                    pltpu.make_async_copy(y_hbm.at[src_v.at[j + NBUF - WS]],
                                          bufs[kk], gsem).start()

        for k in range(WS):
            pltpu.make_async_copy(bufs[0], acc.at[dst_v.at[0]], ssem).wait()

        if with_cnt:
            @pl.when(mine)
            def _():
                ones16 = jnp.ones((16,), jnp.float32)

                @pl.loop(0, ITER)
                def _(i):
                    @pl.loop(0, B // 16)
                    def _(k):
                        v = dst_v[i, pl.ds(k * 16, 16)]
                        row = lax.shift_right_logical(v, 4)
                        col = lax.bitwise_and(v, 15)
                        plsc.addupdate_scatter(cl_v, [row, col], ones16)

                for kk in range(CR // B):
                    pltpu.sync_copy(cl_v.at[pl.ds(kk * B, B)],
                                    scnt.at[idc_v.at[kk]], add=True)

        plsc.subcore_barrier()
        pltpu.sync_copy(acc.at[pl.ds(s * RPT, RPT)],
                        out_hbm.at[c, t, pl.ds(s * RPT, RPT)])
        if with_cnt:
            @pl.when(mine)
            def _():
                pltpu.sync_copy(scnt.at[pl.ds(s * CPT, CPT)],
                                cnt_hbm.at[t, pl.ds(s * CPT, CPT)])


def _make_sc_agg(with_cnt):
    out_type = [jax.ShapeDtypeStruct((NCORE, T, NP, KH), jnp.float32)]
    scratch = [
        pltpu.VMEM((ITER, B), jnp.int32),
        pltpu.VMEM((ITER, B), jnp.int32),
    ] + [pltpu.VMEM((B, KH), jnp.float32) for _ in range(NBUF)] + [
        pltpu.VMEM_SHARED((NP, KH), jnp.float32),
        pltpu.SemaphoreType.DMA,
        pltpu.SemaphoreType.DMA,
    ]
    if with_cnt:
        out_type.append(jax.ShapeDtypeStruct((T, CR, CW), jnp.float32))
        scratch += [
            pltpu.VMEM((CR, CW), jnp.float32),
            pltpu.VMEM((CR // B, B), jnp.int32),
            pltpu.VMEM_SHARED((CR, CW), jnp.float32),
        ]
    cp = pltpu.CompilerParams()
    fields = pltpu.CompilerParams.__dataclass_fields__
    if "use_tc_tiling_on_sc" in fields:
        cp = dataclasses.replace(cp, use_tc_tiling_on_sc=False)
    if with_cnt and "needs_layout_passes" in fields:
        cp = dataclasses.replace(cp, needs_layout_passes=False)
    return pl.kernel(
        functools.partial(_sc_agg_body, with_cnt),
        mesh=_mesh,
        out_type=out_type if with_cnt else out_type[0],
        scratch_types=scratch,
        compiler_params=cp,
    )


_sc_agg_cnt = _make_sc_agg(True)
_sc_agg = _make_sc_agg(False)


BM = 2000
BPT = N // BM


def _mm_body(x_ref, w_ref, o_ref):
    o_ref[0] = jnp.dot(x_ref[...], w_ref[0],
                       preferred_element_type=jnp.float32)


_mm = pl.pallas_call(
    _mm_body,
    grid=(M // BM, NCORE),
    in_specs=[pl.BlockSpec((BM, D), lambda i, h: (i, 0)),
              pl.BlockSpec((1, D, KH), lambda i, h: (h, 0, 0))],
    out_specs=pl.BlockSpec((1, BM, KH), lambda i, h: (h, i, 0)),
    out_shape=jax.ShapeDtypeStruct((NCORE, M, KH), jnp.float32),
)


def _relu_layer(p_ref, cnt_ref, x_ref, wr_ref, bl_ref):
    p = jnp.concatenate((p_ref[0], p_ref[1]), axis=-1)
    recip = 1.0 / jnp.maximum(cnt_ref[...], 1.0)
    return jnp.maximum(
        p * recip + bl_ref[...]
        + jnp.dot(x_ref[...], wr_ref[...], preferred_element_type=jnp.float32),
        0.0)


def _post_body(p_ref, cnt_ref, x_ref, wr_ref, bl_ref, wn_ref, h_ref, y_ref):
    h = _relu_layer(p_ref, cnt_ref, x_ref, wr_ref, bl_ref)
    h_ref[...] = h
    y_ref[0] = jnp.dot(h, wn_ref[0], preferred_element_type=jnp.float32)


_post = pl.pallas_call(
    _post_body,
    grid=(M // BM, NCORE),
    in_specs=[pl.BlockSpec((NCORE, BM, KH), lambda i, h: (0, i, 0)),
              pl.BlockSpec((BM, 1), lambda i, h: (i, 0)),
              pl.BlockSpec((BM, H), lambda i, h: (i, 0)),
              pl.BlockSpec((H, H), lambda i, h: (0, 0)),
              pl.BlockSpec((1, H), lambda i, h: (0, 0)),
              pl.BlockSpec((1, H, KH), lambda i, h: (h, 0, 0))],
    out_specs=[pl.BlockSpec((BM, H), lambda i, h: (i, 0)),
               pl.BlockSpec((1, BM, KH), lambda i, h: (h, i, 0))],
    out_shape=[jax.ShapeDtypeStruct((M, H), jnp.float32),
               jax.ShapeDtypeStruct((NCORE, M, KH), jnp.float32)],
)


def _final_body(p_ref, cnt_ref, x_ref, wr_ref, bl_ref, o_ref):
    i = pl.program_id(0)
    h = _relu_layer(p_ref, cnt_ref, x_ref, wr_ref, bl_ref)

    @pl.when(i == 0)
    def _():
        o_ref[...] = jnp.zeros_like(o_ref)

    rid = lax.broadcasted_iota(jnp.int32, (T, H), 0)
    ps = jnp.sum(h, axis=0, keepdims=True)
    o_ref[...] += jnp.where(rid == i // BPT, jnp.broadcast_to(ps, (T, H)), 0.0)


_final = pl.pallas_call(
    _final_body,
    grid=(M // BM,),
    in_specs=[pl.BlockSpec((NCORE, BM, KH), lambda i: (0, i, 0)),
              pl.BlockSpec((BM, 1), lambda i: (i, 0)),
              pl.BlockSpec((BM, H), lambda i: (i, 0)),
              pl.BlockSpec((H, H), lambda i: (0, 0)),
              pl.BlockSpec((1, H), lambda i: (0, 0))],
    out_specs=pl.BlockSpec((T, H), lambda i: (0, 0)),
    out_shape=jax.ShapeDtypeStruct((T, H), jnp.float32),
)


def _head_body(ps_ref, wq_ref, bq_ref, wk_ref, bk_ref, wv_ref, bv_ref,
               wo_ref, bo_ref, wh1_ref, bh1_ref, wh2_ref, bh2_ref, o_ref):
    seq = ps_ref[...] * jnp.float32(1.0 / N)
    q = jnp.dot(seq, wq_ref[...], preferred_element_type=jnp.float32) + bq_ref[...]
    k = jnp.dot(seq, wk_ref[...], preferred_element_type=jnp.float32) + bk_ref[...]
    v = jnp.dot(seq, wv_ref[...], preferred_element_type=jnp.float32) + bv_ref[...]
    lane = lax.broadcasted_iota(jnp.int32, (T, H), 1)
    scale = jnp.float32(1.0 / (HD ** 0.5))
    o = jnp.zeros((T, H), jnp.float32)
    for h in range(NH):
        m = (lane // HD) == h
        qh = jnp.where(m, q, 0.0)
        kh = jnp.where(m, k, 0.0)
        vh = jnp.where(m, v, 0.0)
        logits = lax.dot_general(qh, kh, (((1,), (1,)), ((), ())),
                                 preferred_element_type=jnp.float32) * scale
        a = jax.nn.softmax(logits, axis=-1)
        o = o + lax.dot_general(a, vh, (((1,), (0,)), ((), ())),
                                preferred_element_type=jnp.float32)
    oo = jnp.dot(o, wo_ref[...], preferred_element_type=jnp.float32) + bo_ref[...]
    last = oo[T - 1:T, :]
    h1 = jnp.maximum(
        jnp.dot(last, wh1_ref[...], preferred_element_type=jnp.float32)
        + bh1_ref[...], 0.0)
    z = jnp.sum(h1 * wh2_ref[...], axis=1, keepdims=True) + bh2_ref[...]
    o_ref[...] = 1.0 / (1.0 + jnp.exp(-z))


_head = pl.pallas_call(
    _head_body,
    out_shape=jax.ShapeDtypeStruct((1, 1), jnp.float32),
)


def _split_cols(w):
    return w.reshape(H, NCORE, KH).transpose(1, 0, 2)


def kernel(xs, edge_indices, Wl1, bl1, Wr1, Wl2, bl2, Wr2, Wl3, bl3, Wr3,
           Wq, bq, Wk, bk, Wv, bv, Wo, bo, Wh1, bh1, Wh2, bh2):
    X0 = xs.reshape(M, D)
    off = (jnp.arange(T, dtype=jnp.int32) * N)[:, None]
    pad = ((0, 0), (0, 0), (0, ECP - EC))
    src = (edge_indices[:, 0, :] + off).reshape(T, NS, EC)
    src = jnp.pad(src, pad).reshape(T, NS, ITER, B)
    src_g = jnp.stack((src, src + M))
    dst = edge_indices[:, 1, :].reshape(T, NS, EC)
    dst = jnp.pad(dst, pad, constant_values=NP - 1)
    dst_g = dst.reshape(T, NS, ITER, B)
    zeros_h = jnp.zeros((RPT, KH), jnp.float32)
    zeros_c = jnp.zeros((CR, CW), jnp.float32)
    idc = jnp.arange(CR, dtype=jnp.int32).reshape(CR // B, B)

    Y1 = _mm(X0, _split_cols(Wl1))
    Y1 = Y1.reshape(NCORE * M, KH)
    P1, CNT = _sc_agg_cnt(Y1, src_g, dst_g, zeros_h, zeros_c, idc)
    cnt = CNT.reshape(T, NP)[:, :N].reshape(M, 1)
    P1 = P1[:, :, :N, :].reshape(NCORE, M, KH)
    X1, Y2 = _post(P1, cnt, X0, Wr1, bl1.reshape(1, H), _split_cols(Wl2))
    P2 = _sc_agg(Y2.reshape(NCORE * M, KH), src_g, dst_g, zeros_h)
    P2 = P2[:, :, :N, :].reshape(NCORE, M, KH)
    X2, Y3 = _post(P2, cnt, X1, Wr2, bl2.reshape(1, H), _split_cols(Wl3))
    P3 = _sc_agg(Y3.reshape(NCORE * M, KH), src_g, dst_g, zeros_h)
    P3 = P3[:, :, :N, :].reshape(NCORE, M, KH)
    PS = _final(P3, cnt, X2, Wr3, bl3.reshape(1, H))
    out = _head(PS, Wq, bq.reshape(1, H), Wk, bk.reshape(1, H),
                Wv, bv.reshape(1, H), Wo, bo.reshape(1, H),
                Wh1, bh1.reshape(1, H // 2), Wh2.reshape(1, H // 2),
                bh2.reshape(1, 1))
    return out.reshape(1)


# speedup vs baseline: 1.0099x; 1.0099x over previous
import dataclasses
import functools

import jax
import jax.numpy as jnp
from jax import lax
from jax.experimental import pallas as pl
from jax.experimental.pallas import tpu as pltpu
from jax.experimental.pallas import tpu_sc as plsc

T = 4
N = 10000
E = 320000
D = 128
H = 128
NH = 4
HD = H // NH
M = T * N

NS = 16
NCORE = 2
KH = H // 2
EC = E // NS
B = 80
ITER = 252
ECP = ITER * B
NBUF = 6
CW = 16
CR = 10240 // CW
NP = 10240
RPT = NP // NS
CPT = CR // NS

_mesh = plsc.VectorSubcoreMesh(core_axis_name="c", subcore_axis_name="s")


def _sc_agg_body(with_cnt, *refs):
    if with_cnt:
        (y_hbm, src_hbm, dst_hbm, z_hbm, zc_hbm, idc_hbm, out_hbm, cnt_hbm,
         src_v, dst_v, *bufs, acc, gsem, cl_v, idc_v, scnt) = refs
    else:
        (y_hbm, src_hbm, dst_hbm, z_hbm, out_hbm,
         src_v, dst_v, *bufs, acc, gsem) = refs

    c = lax.axis_index("c")
    s = lax.axis_index("s")

    if with_cnt:
        pltpu.sync_copy(idc_hbm, idc_v)

    for t in range(T):
        mine = (t // 2) == c
        pltpu.sync_copy(src_hbm.at[c, t, s], src_v)
        pltpu.sync_copy(dst_hbm.at[t, s], dst_v)
        pltpu.sync_copy(z_hbm, acc.at[pl.ds(s * RPT, RPT)])
        if with_cnt:
            @pl.when(mine)
            def _():
                pltpu.sync_copy(zc_hbm, cl_v)
                pltpu.sync_copy(zc_hbm.at[pl.ds(0, CPT)],
                                scnt.at[pl.ds(s * CPT, CPT)])
        plsc.subcore_barrier()

        for k in range(NBUF - 1):
            pltpu.make_async_copy(y_hbm.at[src_v.at[k]], bufs[k],
                                  gsem).start()

        @pl.loop(0, ITER, step=NBUF)
        def _(i):
            for k in range(NBUF):
                j = i + k

                @pl.when(j + NBUF - 1 < ITER)
                def _():
                    kk = (k + NBUF - 1) % NBUF
                    pltpu.make_async_copy(y_hbm.at[src_v.at[j + NBUF - 1]],
                                          bufs[kk], gsem).start()

                pltpu.make_async_copy(y_hbm.at[src_v.at[j]], bufs[k],
                                      gsem).wait()
                pltpu.sync_copy(bufs[k], acc.at[dst_v.at[j]], add=True)

        if with_cnt:
            @pl.when(mine)
            def _():
                ones16 = jnp.ones((16,), jnp.float32)

                @pl.loop(0, ITER)
                def _(i):
                    @pl.loop(0, B // 16)
                    def _(k):
                        v = dst_v[i, pl.ds(k * 16, 16)]
                        row = lax.shift_right_logical(v, 4)
                        col = lax.bitwise_and(v, 15)
                        plsc.addupdate_scatter(cl_v, [row, col], ones16)

                for kk in range(CR // B):
                    pltpu.sync_copy(cl_v.at[pl.ds(kk * B, B)],
                                    scnt.at[idc_v.at[kk]], add=True)

        plsc.subcore_barrier()
        pltpu.sync_copy(acc.at[pl.ds(s * RPT, RPT)],
                        out_hbm.at[c, t, pl.ds(s * RPT, RPT)])
        if with_cnt:
            @pl.when(mine)
            def _():
                pltpu.sync_copy(scnt.at[pl.ds(s * CPT, CPT)],
                                cnt_hbm.at[t, pl.ds(s * CPT, CPT)])


def _make_sc_agg(with_cnt):
    out_type = [jax.ShapeDtypeStruct((NCORE, T, NP, KH), jnp.float32)]
    scratch = [
        pltpu.VMEM((ITER, B), jnp.int32),
        pltpu.VMEM((ITER, B), jnp.int32),
    ] + [pltpu.VMEM((B, KH), jnp.float32) for _ in range(NBUF)] + [
        pltpu.VMEM_SHARED((NP, KH), jnp.float32),
        pltpu.SemaphoreType.DMA,
    ]
    if with_cnt:
        out_type.append(jax.ShapeDtypeStruct((T, CR, CW), jnp.float32))
        scratch += [
            pltpu.VMEM((CR, CW), jnp.float32),
            pltpu.VMEM((CR // B, B), jnp.int32),
            pltpu.VMEM_SHARED((CR, CW), jnp.float32),
        ]
    cp = pltpu.CompilerParams()
    fields = pltpu.CompilerParams.__dataclass_fields__
    if "use_tc_tiling_on_sc" in fields:
        cp = dataclasses.replace(cp, use_tc_tiling_on_sc=False)
    if with_cnt and "needs_layout_passes" in fields:
        cp = dataclasses.replace(cp, needs_layout_passes=False)
    return pl.kernel(
        functools.partial(_sc_agg_body, with_cnt),
        mesh=_mesh,
        out_type=out_type if with_cnt else out_type[0],
        scratch_types=scratch,
        compiler_params=cp,
    )


_sc_agg_cnt = _make_sc_agg(True)
_sc_agg = _make_sc_agg(False)


BM = 2000
BPT = N // BM


def _mm_body(x_ref, w_ref, o_ref):
    o_ref[0] = jnp.dot(x_ref[...], w_ref[0],
                       preferred_element_type=jnp.float32)


_mm = pl.pallas_call(
    _mm_body,
    grid=(M // BM, NCORE),
    in_specs=[pl.BlockSpec((BM, D), lambda i, h: (i, 0)),
              pl.BlockSpec((1, D, KH), lambda i, h: (h, 0, 0))],
    out_specs=pl.BlockSpec((1, BM, KH), lambda i, h: (h, i, 0)),
    out_shape=jax.ShapeDtypeStruct((NCORE, M, KH), jnp.float32),
)


def _relu_layer(p_ref, cnt_ref, x_ref, wr_ref, bl_ref):
    p = jnp.concatenate((p_ref[0], p_ref[1]), axis=-1)
    recip = 1.0 / jnp.maximum(cnt_ref[...], 1.0)
    return jnp.maximum(
        p * recip + bl_ref[...]
        + jnp.dot(x_ref[...], wr_ref[...], preferred_element_type=jnp.float32),
        0.0)


def _post_body(p_ref, cnt_ref, x_ref, wr_ref, bl_ref, wn_ref, h_ref, y_ref):
    h = _relu_layer(p_ref, cnt_ref, x_ref, wr_ref, bl_ref)
    h_ref[...] = h
    y_ref[0] = jnp.dot(h, wn_ref[0], preferred_element_type=jnp.float32)


_post = pl.pallas_call(
    _post_body,
    grid=(M // BM, NCORE),
    in_specs=[pl.BlockSpec((NCORE, BM, KH), lambda i, h: (0, i, 0)),
              pl.BlockSpec((BM, 1), lambda i, h: (i, 0)),
              pl.BlockSpec((BM, H), lambda i, h: (i, 0)),
              pl.BlockSpec((H, H), lambda i, h: (0, 0)),
              pl.BlockSpec((1, H), lambda i, h: (0, 0)),
              pl.BlockSpec((1, H, KH), lambda i, h: (h, 0, 0))],
    out_specs=[pl.BlockSpec((BM, H), lambda i, h: (i, 0)),
               pl.BlockSpec((1, BM, KH), lambda i, h: (h, i, 0))],
    out_shape=[jax.ShapeDtypeStruct((M, H), jnp.float32),
               jax.ShapeDtypeStruct((NCORE, M, KH), jnp.float32)],
)


def _final_body(p_ref, cnt_ref, x_ref, wr_ref, bl_ref, o_ref):
    i = pl.program_id(0)
    h = _relu_layer(p_ref, cnt_ref, x_ref, wr_ref, bl_ref)

    @pl.when(i == 0)
    def _():
        o_ref[...] = jnp.zeros_like(o_ref)

    rid = lax.broadcasted_iota(jnp.int32, (T, H), 0)
    ps = jnp.sum(h, axis=0, keepdims=True)
    o_ref[...] += jnp.where(rid == i // BPT, jnp.broadcast_to(ps, (T, H)), 0.0)


_final = pl.pallas_call(
    _final_body,
    grid=(M // BM,),
    in_specs=[pl.BlockSpec((NCORE, BM, KH), lambda i: (0, i, 0)),
              pl.BlockSpec((BM, 1), lambda i: (i, 0)),
              pl.BlockSpec((BM, H), lambda i: (i, 0)),
              pl.BlockSpec((H, H), lambda i: (0, 0)),
              pl.BlockSpec((1, H), lambda i: (0, 0))],
    out_specs=pl.BlockSpec((T, H), lambda i: (0, 0)),
    out_shape=jax.ShapeDtypeStruct((T, H), jnp.float32),
)


def _head_body(ps_ref, wq_ref, bq_ref, wk_ref, bk_ref, wv_ref, bv_ref,
               wo_ref, bo_ref, wh1_ref, bh1_ref, wh2_ref, bh2_ref, o_ref):
    seq = ps_ref[...] * jnp.float32(1.0 / N)
    q = jnp.dot(seq, wq_ref[...], preferred_element_type=jnp.float32) + bq_ref[...]
    k = jnp.dot(seq, wk_ref[...], preferred_element_type=jnp.float32) + bk_ref[...]
    v = jnp.dot(seq, wv_ref[...], preferred_element_type=jnp.float32) + bv_ref[...]
    lane = lax.broadcasted_iota(jnp.int32, (T, H), 1)
    scale = jnp.float32(1.0 / (HD ** 0.5))
    o = jnp.zeros((T, H), jnp.float32)
    for h in range(NH):
        m = (lane // HD) == h
        qh = jnp.where(m, q, 0.0)
        kh = jnp.where(m, k, 0.0)
        vh = jnp.where(m, v, 0.0)
        logits = lax.dot_general(qh, kh, (((1,), (1,)), ((), ())),
                                 preferred_element_type=jnp.float32) * scale
        a = jax.nn.softmax(logits, axis=-1)
        o = o + lax.dot_general(a, vh, (((1,), (0,)), ((), ())),
                                preferred_element_type=jnp.float32)
    oo = jnp.dot(o, wo_ref[...], preferred_element_type=jnp.float32) + bo_ref[...]
    last = oo[T - 1:T, :]
    h1 = jnp.maximum(
        jnp.dot(last, wh1_ref[...], preferred_element_type=jnp.float32)
        + bh1_ref[...], 0.0)
    z = jnp.sum(h1 * wh2_ref[...], axis=1, keepdims=True) + bh2_ref[...]
    o_ref[...] = 1.0 / (1.0 + jnp.exp(-z))


_head = pl.pallas_call(
    _head_body,
    out_shape=jax.ShapeDtypeStruct((1, 1), jnp.float32),
)


def _split_cols(w):
    return w.reshape(H, NCORE, KH).transpose(1, 0, 2)


def kernel(xs, edge_indices, Wl1, bl1, Wr1, Wl2, bl2, Wr2, Wl3, bl3, Wr3,
           Wq, bq, Wk, bk, Wv, bv, Wo, bo, Wh1, bh1, Wh2, bh2):
    X0 = xs.reshape(M, D)
    off = (jnp.arange(T, dtype=jnp.int32) * N)[:, None]
    pad = ((0, 0), (0, 0), (0, ECP - EC))
    src = (edge_indices[:, 0, :] + off).reshape(T, NS, EC)
    src = jnp.pad(src, pad).reshape(T, NS, ITER, B)
    src_g = jnp.stack((src, src + M))
    dst = edge_indices[:, 1, :].reshape(T, NS, EC)
    dst = jnp.pad(dst, pad, constant_values=NP - 1)
    dst_g = dst.reshape(T, NS, ITER, B)
    zeros_h = jnp.zeros((RPT, KH), jnp.float32)
    zeros_c = jnp.zeros((CR, CW), jnp.float32)
    idc = jnp.arange(CR, dtype=jnp.int32).reshape(CR // B, B)

    Y1 = _mm(X0, _split_cols(Wl1))
    Y1 = Y1.reshape(NCORE * M, KH)
    P1, CNT = _sc_agg_cnt(Y1, src_g, dst_g, zeros_h, zeros_c, idc)
    cnt = CNT.reshape(T, NP)[:, :N].reshape(M, 1)
    P1 = P1[:, :, :N, :].reshape(NCORE, M, KH)
    X1, Y2 = _post(P1, cnt, X0, Wr1, bl1.reshape(1, H), _split_cols(Wl2))
    P2 = _sc_agg(Y2.reshape(NCORE * M, KH), src_g, dst_g, zeros_h)
    P2 = P2[:, :, :N, :].reshape(NCORE, M, KH)
    X2, Y3 = _post(P2, cnt, X1, Wr2, bl2.reshape(1, H), _split_cols(Wl3))
    P3 = _sc_agg(Y3.reshape(NCORE * M, KH), src_g, dst_g, zeros_h)
    P3 = P3[:, :, :N, :].reshape(NCORE, M, KH)
    PS = _final(P3, cnt, X2, Wr3, bl3.reshape(1, H))
    out = _head(PS, Wq, bq.reshape(1, H), Wk, bk.reshape(1, H),
                Wv, bv.reshape(1, H), Wo, bo.reshape(1, H),
                Wh1, bh1.reshape(1, H // 2), Wh2.reshape(1, H // 2),
                bh2.reshape(1, 1))
    return out.reshape(1)
